# baseline (speedup 1.0000x reference)
import functools

import jax
import jax.numpy as jnp
from jax import lax
from jax.experimental import pallas as pl
from jax.experimental.pallas import tpu as pltpu
from jax.experimental.pallas import tpu_sc as plsc

L = 200
D = 64
OUT = 2
NCHUNK = 2
CHUNK = L // NCHUNK


def _dan_kernel(x_hbm, table_hbm, w_hbm, b_hbm, out_hbm,
                idx_v, rows_v, w_v, b_v, out_v, sem):
    is_w0 = jnp.logical_and(lax.axis_index("c") == 0, lax.axis_index("s") == 0)

    @pl.when(is_w0)
    def _():
        pltpu.sync_copy(x_hbm, idx_v)
        pltpu.sync_copy(w_hbm, w_v)
        pltpu.sync_copy(b_hbm, b_v)
        cps = [
            pltpu.async_copy(
                table_hbm.at[idx_v.at[j]],
                rows_v.at[pl.ds(j * CHUNK, CHUNK)],
                sem,
            )
            for j in range(NCHUNK)
        ]
        for cp in cps:
            cp.wait()

        zero = jnp.zeros((16,), jnp.float32)

        def body(i, acc):
            a0, a1, a2, a3 = acc
            return (a0 + rows_v[i, pl.ds(0, 16)],
                    a1 + rows_v[i, pl.ds(16, 16)],
                    a2 + rows_v[i, pl.ds(32, 16)],
                    a3 + rows_v[i, pl.ds(48, 16)])

        a0, a1, a2, a3 = lax.fori_loop(0, L, body, (zero, zero, zero, zero))

        inv_n = jnp.float32(1.0 / L)
        m0, m1, m2, m3 = a0 * inv_n, a1 * inv_n, a2 * inv_n, a3 * inv_n

        p0 = (w_v[0, pl.ds(0, 16)] * m0 + w_v[0, pl.ds(16, 16)] * m1
              + w_v[0, pl.ds(32, 16)] * m2 + w_v[0, pl.ds(48, 16)] * m3)
        p1 = (w_v[1, pl.ds(0, 16)] * m0 + w_v[1, pl.ds(16, 16)] * m1
              + w_v[1, pl.ds(32, 16)] * m2 + w_v[1, pl.ds(48, 16)] * m3)

        lane = lax.iota(jnp.int32, 16)
        oh0 = lane == 0
        oh01 = lane < OUT

        s0 = jnp.full((16,), jnp.sum(p0), jnp.float32)
        s1 = jnp.full((16,), jnp.sum(p1), jnp.float32)
        bv = b_v[...]
        lv = jnp.where(oh01, jnp.where(oh0, s0, s1) + bv,
                       jnp.float32(-100.0))

        mx = jnp.full(
            (16,),
            jnp.max(jnp.where(oh01, lv, jnp.float32(-1e30))),
            jnp.float32,
        )
        dv = lv - mx
        ev = jnp.exp(dv)
        sv = jnp.full((16,), jnp.sum(ev), jnp.float32)

        t = sv - 1.0
        y = t * (1.0 - t * (0.5 - t * (1.0 / 3.0 - 0.25 * t)))
        for _ in range(3):
            y = y + sv * jnp.exp(-y) - 1.0

        out_v[...] = jnp.where(oh01, dv - y, jnp.float32(0.0))
        pltpu.sync_copy(out_v, out_hbm)


@functools.partial(jax.jit, static_argnames=())
def _dan_call(x2, table, w, bpad):
    mesh = plsc.VectorSubcoreMesh(core_axis_name="c", subcore_axis_name="s")
    f = functools.partial(
        pl.kernel,
        out_type=jax.ShapeDtypeStruct((16,), jnp.float32),
        mesh=mesh,
        compiler_params=pltpu.CompilerParams(
            needs_layout_passes=False, use_tc_tiling_on_sc=False
        ),
        scratch_types=[
            pltpu.VMEM((NCHUNK, CHUNK), jnp.int32),
            pltpu.VMEM((L, D), jnp.float32),
            pltpu.VMEM((OUT, D), jnp.float32),
            pltpu.VMEM((16,), jnp.float32),
            pltpu.VMEM((16,), jnp.float32),
            pltpu.SemaphoreType.DMA,
        ],
    )(_dan_kernel)
    return f(x2, table, w, bpad)


def kernel(x, table, W, b):
    x2 = x.reshape(NCHUNK, CHUNK)
    bpad = jnp.zeros((16,), jnp.float32).at[:OUT].set(b)
    res = _dan_call(x2, table, W, bpad)
    return res[:OUT]


# speedup vs baseline: 39.9524x; 39.9524x over previous
import jax
import jax.numpy as jnp
from jax.experimental import pallas as pl
from jax.experimental.pallas import tpu as pltpu

L = 200
D = 64
OUT = 2
WIN = 128
TPB = 8
GRID = L // TPB


def _dan_body(x_sref, *refs):
    blocks = refs[:TPB]
    w_ref, b_ref, out_ref, acc = refs[TPB:]
    j = pl.program_id(0)

    @pl.when(j == 0)
    def _():
        acc[...] = jnp.zeros((D, WIN), jnp.float32)

    lane = jax.lax.broadcasted_iota(jnp.int32, (D, WIN), 1)
    a = acc[...]
    for k in range(TPB):
        sub = x_sref[j * TPB + k] & (WIN - 1)
        a = a + jnp.where(lane == sub, blocks[k][...], jnp.float32(0.0))
    acc[...] = a

    @pl.when(j == GRID - 1)
    def _():
        mean = jnp.sum(a, axis=1) * jnp.float32(1.0 / L)
        logits = w_ref[...] @ mean + b_ref[...]
        mx = jnp.max(logits)
        z = logits - mx
        out_ref[...] = z - jnp.log(jnp.sum(jnp.exp(z)))


def _win_spec(k):
    return pl.BlockSpec((D, WIN), lambda j, xr: (0, xr[j * TPB + k] >> 7))


@jax.jit
def _dan_call(x, t2, w, b):
    return pl.pallas_call(
        _dan_body,
        grid_spec=pltpu.PrefetchScalarGridSpec(
            num_scalar_prefetch=1,
            grid=(GRID,),
            in_specs=[_win_spec(k) for k in range(TPB)]
            + [
                pl.BlockSpec((OUT, D), lambda j, xr: (0, 0)),
                pl.BlockSpec((OUT,), lambda j, xr: (0,)),
            ],
            out_specs=pl.BlockSpec((OUT,), lambda j, xr: (0,)),
            scratch_shapes=[pltpu.VMEM((D, WIN), jnp.float32)],
        ),
        out_shape=jax.ShapeDtypeStruct((OUT,), jnp.float32),
        compiler_params=pltpu.CompilerParams(
            dimension_semantics=("arbitrary",),
        ),
    )(x, *([t2] * TPB), w, b)


def kernel(x, table, W, b):
    t2 = table.T
    return _dan_call(x, t2, W, b)


# speedup vs baseline: 71.6295x; 1.7929x over previous
import jax
import jax.numpy as jnp
from jax.experimental import pallas as pl
from jax.experimental.pallas import tpu as pltpu

L = 200
D = 64
OUT = 2
WIN = 128
TPB = 25
GRID = L // TPB


def _dan_body(x_sref, *refs):
    blocks = refs[:TPB]
    w_ref, b_ref, out_ref, acc = refs[TPB:]
    j = pl.program_id(0)

    @pl.when(j == 0)
    def _():
        acc[...] = jnp.zeros((D, WIN), jnp.float32)

    lane = jax.lax.broadcasted_iota(jnp.int32, (D, WIN), 1)
    a = acc[...]
    for k in range(TPB):
        sub = x_sref[j * TPB + k] & (WIN - 1)
        a = a + jnp.where(lane == sub, blocks[k][...], jnp.float32(0.0))
    acc[...] = a

    @pl.when(j == GRID - 1)
    def _():
        mean = jnp.sum(a, axis=1) * jnp.float32(1.0 / L)
        logits = w_ref[...] @ mean + b_ref[...]
        mx = jnp.max(logits)
        z = logits - mx
        out_ref[...] = z - jnp.log(jnp.sum(jnp.exp(z)))


def _win_spec(k):
    return pl.BlockSpec((D, WIN), lambda j, xr: (0, xr[j * TPB + k] >> 7))


@jax.jit
def _dan_call(x, t2, w, b):
    return pl.pallas_call(
        _dan_body,
        grid_spec=pltpu.PrefetchScalarGridSpec(
            num_scalar_prefetch=1,
            grid=(GRID,),
            in_specs=[_win_spec(k) for k in range(TPB)]
            + [
                pl.BlockSpec((OUT, D), lambda j, xr: (0, 0)),
                pl.BlockSpec((OUT,), lambda j, xr: (0,)),
            ],
            out_specs=pl.BlockSpec((OUT,), lambda j, xr: (0,)),
            scratch_shapes=[pltpu.VMEM((D, WIN), jnp.float32)],
        ),
        out_shape=jax.ShapeDtypeStruct((OUT,), jnp.float32),
        compiler_params=pltpu.CompilerParams(
            dimension_semantics=("arbitrary",),
        ),
    )(x, *([t2] * TPB), w, b)


def kernel(x, table, W, b):
    t2 = table.T
    return _dan_call(x, t2, W, b)
